# baseline (speedup 1.0000x reference)
import functools
import jax
import jax.numpy as jnp
from jax import lax
from jax.experimental import pallas as pl
from jax.experimental.pallas import tpu as pltpu
from jax.experimental.pallas import tpu_sc as plsc


def _vq_body(x_ref, w_in_t_ref, b_in_ref, et2_ref, embed_ref,
             w_out_t_ref, b_out_ref, out_ref):
    z = jnp.dot(x_ref[...], w_in_t_ref[...],
                preferred_element_type=jnp.float32) + b_in_ref[...]
    et2 = et2_ref[...]
    s2 = jnp.dot(z, et2, preferred_element_type=jnp.float32)
    e_sq = 0.25 * jnp.sum(et2 * et2, axis=0, keepdims=True)
    scores = s2 - e_sq
    idx = jnp.argmax(scores, axis=1).astype(jnp.int32)
    iota = lax.broadcasted_iota(jnp.int32, scores.shape, 1)
    onehot = jnp.where(iota == idx[:, None], 1.0, 0.0)
    q = jnp.dot(onehot, embed_ref[...],
                preferred_element_type=jnp.float32)
    out_ref[...] = jnp.dot(q, w_out_t_ref[...],
                           preferred_element_type=jnp.float32) + b_out_ref[...]


def _argmin_body(x_ref, w_in_t_ref, b_in_ref, et2_ref, embed_ref,
                 w_out_t_ref, b_out_ref, idx_ref, eout_ref):
    z = jnp.dot(x_ref[...], w_in_t_ref[...],
                preferred_element_type=jnp.float32) + b_in_ref[...]
    et2 = et2_ref[...]
    s2 = jnp.dot(z, et2, preferred_element_type=jnp.float32)
    e_sq = 0.25 * jnp.sum(et2 * et2, axis=0, keepdims=True)
    scores = s2 - e_sq
    idx = jnp.argmax(scores, axis=1).astype(jnp.int32)
    idx_ref[...] = idx.reshape(idx_ref.shape)

    @pl.when(pl.program_id(0) == 0)
    def _():
        eout_ref[...] = jnp.dot(embed_ref[...], w_out_t_ref[...],
                                preferred_element_type=jnp.float32) + b_out_ref[...]


def _make_sc_gather(bt, dim, n_workers, nc, chunk):
    b_per_w = bt // n_workers
    n_chunks = b_per_w // chunk
    mesh = plsc.VectorSubcoreMesh(core_axis_name="c", subcore_axis_name="s")

    @functools.partial(
        pl.kernel,
        out_type=jax.ShapeDtypeStruct((bt, dim), jnp.float32),
        mesh=mesh,
        scratch_types=[
            pltpu.VMEM((b_per_w,), jnp.int32),
            pltpu.VMEM((2, chunk, dim), jnp.float32),
            pltpu.SemaphoreType.DMA,
            pltpu.SemaphoreType.DMA,
            pltpu.SemaphoreType.DMA,
            pltpu.SemaphoreType.DMA,
        ],
    )
    def sc_gather(idx_hbm, table_hbm, out_hbm, idx_v, rows_v, g0, g1, w0, w1):
        wid = lax.axis_index("s") * nc + lax.axis_index("c")
        base = wid * b_per_w
        gsem = (g0, g1)
        wsem = (w0, w1)
        pltpu.sync_copy(idx_hbm.at[pl.ds(base, b_per_w)], idx_v)

        def fire_gather(c):
            b = c % 2
            return pltpu.async_copy(
                table_hbm.at[idx_v.at[pl.ds(c * chunk, chunk)]],
                rows_v.at[b], gsem[b])

        def fire_write(c):
            b = c % 2
            return pltpu.async_copy(
                rows_v.at[b], out_hbm.at[pl.ds(base + c * chunk, chunk)],
                wsem[b])

        gh = {0: fire_gather(0)}
        wh = {}
        for c in range(1, n_chunks):
            if c >= 2:
                wh[c - 2].wait()
            gh[c] = fire_gather(c)
            gh[c - 1].wait()
            wh[c - 1] = fire_write(c - 1)
        gh[n_chunks - 1].wait()
        wh[n_chunks - 1] = fire_write(n_chunks - 1)
        wh[n_chunks - 2].wait()
        wh[n_chunks - 1].wait()

    return sc_gather


@jax.jit
def kernel(x, W_in, b_in, W_out, b_out, embed):
    b, t, dim = x.shape
    cdim, _ = W_in.shape
    k = embed.shape[0]
    bt = b * t
    flat = x.reshape(bt, dim)
    r = 4096
    s = 8192

    w_in_t = W_in.T
    b_in_row = b_in.reshape(1, cdim)
    et2 = 2.0 * embed.T
    w_out_t = W_out.T
    b_out_row = b_out.reshape(1, dim)

    common_in_specs = [
        pl.BlockSpec((r, dim), lambda i: (i, 0)),
        pl.BlockSpec((dim, cdim), lambda i: (0, 0)),
        pl.BlockSpec((1, cdim), lambda i: (0, 0)),
        pl.BlockSpec((cdim, k), lambda i: (0, 0)),
        pl.BlockSpec((k, cdim), lambda i: (0, 0)),
        pl.BlockSpec((cdim, dim), lambda i: (0, 0)),
        pl.BlockSpec((1, dim), lambda i: (0, 0)),
    ]
    operands = (w_in_t, b_in_row, et2, embed, w_out_t, b_out_row)

    idx3, e_out = pl.pallas_call(
        _argmin_body,
        grid=(s // r,),
        in_specs=common_in_specs,
        out_specs=[
            pl.BlockSpec((1, 1, r), lambda i: (i, 0, 0)),
            pl.BlockSpec((k, dim), lambda i: (0, 0)),
        ],
        out_shape=[
            jax.ShapeDtypeStruct((s // r, 1, r), jnp.int32),
            jax.ShapeDtypeStruct((k, dim), jnp.float32),
        ],
    )(flat[:s], *operands)

    info = plsc.get_sparse_core_info()
    n_workers = info.num_cores * info.num_subcores
    out_a = _make_sc_gather(s, dim, n_workers, info.num_cores, 64)(
        idx3.reshape(s), e_out)

    n_head = s // r
    out_big = pl.pallas_call(
        _vq_body,
        grid=((bt - s) // r,),
        in_specs=common_in_specs,
        out_specs=pl.BlockSpec((r, dim), lambda i: (i + n_head, 0)),
        out_shape=jax.ShapeDtypeStruct((bt, dim), jnp.float32),
    )(flat[s:], *operands)

    out = lax.dynamic_update_slice(out_big, out_a, (0, 0))
    return out.reshape(b, t, dim)


# speedup vs baseline: 1.9495x; 1.9495x over previous
import functools
import jax
import jax.numpy as jnp
from jax import lax
from jax.experimental import pallas as pl
from jax.experimental.pallas import tpu as pltpu


def _vq_body(x_ref, w_in_t_ref, b_in_ref, et2_ref, embed_ref,
             w_out_t_ref, b_out_ref, out_ref):
    z = jnp.dot(x_ref[...], w_in_t_ref[...],
                preferred_element_type=jnp.float32) + b_in_ref[...]
    et2 = et2_ref[...]
    s2 = jnp.dot(z, et2, preferred_element_type=jnp.float32)
    e_sq = 0.25 * jnp.sum(et2 * et2, axis=0, keepdims=True)
    scores = s2 - e_sq
    idx = jnp.argmax(scores, axis=1).astype(jnp.int32)
    iota = lax.broadcasted_iota(jnp.int32, scores.shape, 1)
    onehot = jnp.where(iota == idx[:, None], 1.0, 0.0)
    q = jnp.dot(onehot, embed_ref[...],
                preferred_element_type=jnp.float32)
    out_ref[...] = jnp.dot(q, w_out_t_ref[...],
                           preferred_element_type=jnp.float32) + b_out_ref[...]


@jax.jit
def kernel(x, W_in, b_in, W_out, b_out, embed):
    b, t, dim = x.shape
    cdim, _ = W_in.shape
    k = embed.shape[0]
    bt = b * t
    flat = x.reshape(bt, dim)
    r = 4096
    grid = (bt // r,)

    out = pl.pallas_call(
        _vq_body,
        grid=grid,
        in_specs=[
            pl.BlockSpec((r, dim), lambda i: (i, 0)),
            pl.BlockSpec((dim, cdim), lambda i: (0, 0)),
            pl.BlockSpec((1, cdim), lambda i: (0, 0)),
            pl.BlockSpec((cdim, k), lambda i: (0, 0)),
            pl.BlockSpec((k, cdim), lambda i: (0, 0)),
            pl.BlockSpec((cdim, dim), lambda i: (0, 0)),
            pl.BlockSpec((1, dim), lambda i: (0, 0)),
        ],
        out_specs=pl.BlockSpec((r, dim), lambda i: (i, 0)),
        out_shape=jax.ShapeDtypeStruct((bt, dim), jnp.float32),
    )(flat, W_in.T, b_in.reshape(1, cdim), 2.0 * embed.T, embed,
      W_out.T, b_out.reshape(1, dim))
    return out.reshape(b, t, dim)
